# baseline (speedup 1.0000x reference)
import functools

import jax
import jax.numpy as jnp
from jax import lax
from jax.experimental import pallas as pl
from jax.experimental.pallas import tpu as pltpu
from jax.experimental.pallas import tpu_sc as plsc

C = 1000
CP = 1024
N = 51200
NC, NS, L = 2, 16, 16
NW = NC * NS
VB = C // 8
NSB = N // 128
KMAX = 13
SMAX = KMAX * 128
SFULL = 12 * 128
LZW = 64


def _vector_mesh():
    return plsc.VectorSubcoreMesh(core_axis_name="c", subcore_axis_name="s")


def _prep_body(t_ref, lz_ref, tt_ref, tv_ref):
    x = t_ref[...]
    m = jnp.max(x, axis=1)
    s = jnp.sum(jnp.exp(x - m[:, None]), axis=1)
    lz_ref[...] = m + jnp.log(s)
    tt_ref[...] = x.T
    xp = jnp.concatenate([x, jnp.zeros((C, CP - C), jnp.float32)], axis=1)
    tv_ref[...] = xp.reshape(C * 8, 128)


def _prep(table):
    return pl.pallas_call(
        _prep_body,
        out_shape=(
            jax.ShapeDtypeStruct((C,), jnp.float32),
            jax.ShapeDtypeStruct((C, C), jnp.float32),
            jax.ShapeDtypeStruct((C * 8, 128), jnp.float32),
        ),
    )(table)


def _gather_cm(table_t, tview, rowlz, idx_flat, tgt_flat):
    @functools.partial(
        pl.kernel,
        out_type=(
            jax.ShapeDtypeStruct((C, N), jnp.float32),
            jax.ShapeDtypeStruct((NW, L), jnp.float32),
        ),
        mesh=_vector_mesh(),
        compiler_params=pltpu.CompilerParams(needs_layout_passes=False),
        scratch_types=[
            pltpu.VMEM((SMAX,), jnp.int32),
            pltpu.VMEM((SMAX,), jnp.int32),
            pltpu.VMEM((SMAX,), jnp.int32),
            pltpu.VMEM((SMAX,), jnp.float32),
            pltpu.VMEM((L,), jnp.float32),
            pltpu.VMEM((8, C), jnp.float32),
            pltpu.VMEM((8, C), jnp.float32),
            pltpu.VMEM((8, SMAX), jnp.float32),
            pltpu.VMEM((8, SMAX), jnp.float32),
            pltpu.VMEM((128, 128), jnp.float32),
            pltpu.VMEM((128, 128), jnp.float32),
            pltpu.SemaphoreType.DMA,
            pltpu.SemaphoreType.DMA,
            pltpu.SemaphoreType.DMA,
            pltpu.SemaphoreType.DMA,
            pltpu.SemaphoreType.DMA,
            pltpu.SemaphoreType.DMA,
            pltpu.SemaphoreType.DMA,
        ],
    )
    def k(tt_hbm, tv_hbm, lz_hbm, idx_hbm, tgt_hbm, out_hbm, parts_hbm,
          idx_v, tgt_v, rowp_v, lz_v, acc_v, tr0, tr1, ob0, ob1, pk0, pk1,
          t0, t1, w0, w1, p0, p1, lzsem):
        wid = lax.axis_index("s") * NC + lax.axis_index("c")
        tr = (tr0, tr1)
        ob = (ob0, ob1)
        pk = (pk0, pk1)
        tsem = (t0, t1)
        wsem = (w0, w1)
        psem = (p0, p1)
        iota = jnp.arange(L, dtype=jnp.int32)

        def bid(kk):
            return wid + NW * kk

        @pl.loop(SFULL, SMAX, step=L)
        def _(j):
            idx_v[pl.ds(j, L)] = jnp.zeros((L,), jnp.int32)
            tgt_v[pl.ds(j, L)] = jnp.zeros((L,), jnp.int32)
        @pl.loop(0, KMAX)
        def _(kk):
            @pl.when(bid(kk) < NSB)
            def _():
                pltpu.sync_copy(idx_hbm.at[pl.ds(bid(kk) * 128, 128)],
                                idx_v.at[pl.ds(kk * 128, 128)])
                pltpu.sync_copy(tgt_hbm.at[pl.ds(bid(kk) * 128, 128)],
                                tgt_v.at[pl.ds(kk * 128, 128)])

        @pl.loop(0, SMAX, step=LZW)
        def _(j):
            sl = pl.ds(j, LZW)
            pltpu.make_async_copy(
                lz_hbm.at[idx_v.at[sl]], lz_v.at[sl], lzsem).start()

        @pl.loop(0, SMAX, step=L)
        def _(j):
            sl = pl.ds(j, L)
            rowp_v[sl] = (idx_v[sl] << 3) + (tgt_v[sl] >> 7)

        acc_v[...] = jnp.zeros((L,), jnp.float32)

        def pk_start(j, b):
            pltpu.make_async_copy(
                tv_hbm.at[rowp_v.at[pl.ds(j * 128, 128)]], pk[b], psem[b]
            ).start()

        def pk_wait(j, b):
            pltpu.make_async_copy(
                tv_hbm.at[rowp_v.at[pl.ds(j * 128, 128)]], pk[b], psem[b]
            ).wait()

        pk_start(0, 0)
        for j in range(KMAX):
            b = j % 2
            if j + 1 < KMAX:
                pk_start(j + 1, (j + 1) % 2)
            pk_wait(j, b)

            def _extract():
                for g in range(8):
                    colp = tgt_v[pl.ds(j * 128 + g * L, L)] & 127
                    vals = plsc.load_gather(pk[b], [iota + g * L, colp])
                    acc_v[...] = acc_v[...] - vals
            if j == KMAX - 1:
                @pl.when(wid < 16)
                def _():
                    _extract()
            else:
                _extract()

        def tr_load_start(v, b):
            pltpu.make_async_copy(
                tt_hbm.at[pl.ds(8 * v, 8)], tr[b], tsem[b]).start()

        def tr_load_wait(v, b):
            pltpu.make_async_copy(
                tt_hbm.at[pl.ds(8 * v, 8)], tr[b], tsem[b]).wait()

        def tile_write_cp(v, kk, b):
            return pltpu.make_async_copy(
                ob[b].at[:, pl.ds(kk * 128, 128)],
                out_hbm.at[pl.ds(8 * v, 8), pl.ds(bid(kk) * 128, 128)],
                wsem[b])

        def wait_writes(v, b):
            @pl.loop(0, KMAX)
            def _(kk):
                @pl.when(bid(kk) < NSB)
                def _():
                    tile_write_cp(v, kk, b).wait()

        def compute_v(v, b):
            @pl.loop(0, KMAX)
            def _(kk):
                @pl.when(bid(kk) < NSB)
                def _():
                    for g in range(8):
                        sidx = idx_v[pl.ds(kk * 128 + g * L, L)]
                        for c in range(8):
                            cvec = jnp.full((L,), c, jnp.int32)
                            vals = plsc.load_gather(tr[b], [cvec, sidx])
                            ob[b][c, pl.ds(kk * 128 + g * L, L)] = vals
                    tile_write_cp(v, kk, b).start()

        tr_load_start(0, 0)
        tr_load_start(1, 1)

        @pl.loop(0, VB - 1, step=2)
        def _(v0):
            for b in range(2):
                v = v0 + b

                @pl.when(v >= 2)
                def _():
                    wait_writes(v - 2, b)
                tr_load_wait(v, b)
                compute_v(v, b)

                @pl.when(v + 2 < VB)
                def _():
                    tr_load_start(v + 2, b)

        lv = VB - 1
        wait_writes(lv - 2, 0)
        tr_load_wait(lv, 0)
        compute_v(lv, 0)

        @pl.loop(0, SMAX, step=LZW)
        def _(j):
            sl = pl.ds(j, LZW)
            pltpu.make_async_copy(
                lz_hbm.at[idx_v.at[sl]], lz_v.at[sl], lzsem).wait()

        @pl.loop(0, SFULL, step=L)
        def _(j):
            acc_v[...] = acc_v[...] + lz_v[pl.ds(j, L)]

        @pl.when(wid < 16)
        def _():
            @pl.loop(SFULL, SMAX, step=L)
            def _(j):
                acc_v[...] = acc_v[...] + lz_v[pl.ds(j, L)]

        pltpu.sync_copy(acc_v, parts_hbm.at[wid])
        wait_writes(lv - 1, 1)
        wait_writes(lv, 0)

    return k(table_t, tview, rowlz, idx_flat, tgt_flat)


def _reduce_body(p_ref, o_ref):
    o_ref[...] = (jnp.sum(p_ref[...]) / N).reshape(1, 1)


def _reduce_loss(parts):
    return pl.pallas_call(
        _reduce_body,
        out_shape=jax.ShapeDtypeStruct((1, 1), jnp.float32),
    )(parts)


def kernel(idx, targets, table):
    idx_flat = idx.reshape(-1).astype(jnp.int32)
    tgt_flat = targets.reshape(-1).astype(jnp.int32)
    rowlz, table_t, tview = _prep(table)
    out_cm, parts = _gather_cm(table_t, tview, rowlz, idx_flat, tgt_flat)
    logits2 = out_cm.T
    loss = _reduce_loss(parts)
    return (logits2, loss[0, 0])


# speedup vs baseline: 2.0717x; 2.0717x over previous
import functools

import jax
import jax.numpy as jnp
from jax import lax
from jax.experimental import pallas as pl
from jax.experimental.pallas import tpu as pltpu
from jax.experimental.pallas import tpu_sc as plsc

C = 1000
CP = 1024
TAIL0 = 896
TAILW = C - TAIL0
N = 51200
NSPLIT = 1
NH = N // NSPLIT
NC, NS, L = 2, 16, 16
NW = NC * NS
PER_W = NH // NW
GW = 32
LZW = 80
TBLK = 1024


def _vector_mesh():
    return plsc.VectorSubcoreMesh(core_axis_name="c", subcore_axis_name="s")


def _prep_body(t_ref, lz_ref, pad_ref):
    x = t_ref[...]
    m = jnp.max(x, axis=1)
    s = jnp.sum(jnp.exp(x - m[:, None]), axis=1)
    lz_ref[...] = m + jnp.log(s)
    pad_ref[...] = jnp.concatenate(
        [x, jnp.zeros((C, CP - C), jnp.float32)], axis=1)


def _prep(table):
    return pl.pallas_call(
        _prep_body,
        out_shape=(
            jax.ShapeDtypeStruct((C,), jnp.float32),
            jax.ShapeDtypeStruct((C, CP), jnp.float32),
        ),
    )(table)


def _gather_and_parts(table_pad, rowlz, idx_flat, tgt_flat, split):
    n_chunks = PER_W // GW
    split_base = split * NH

    @functools.partial(
        pl.kernel,
        out_type=(
            jax.ShapeDtypeStruct((NH, C), jnp.float32),
            jax.ShapeDtypeStruct((NW, L), jnp.float32),
        ),
        mesh=_vector_mesh(),
        compiler_params=pltpu.CompilerParams(needs_layout_passes=False),
        scratch_types=[
            pltpu.VMEM((PER_W,), jnp.int32),
            pltpu.VMEM((PER_W,), jnp.int32),
            pltpu.VMEM((PER_W,), jnp.float32),
            pltpu.VMEM((L,), jnp.float32),
            pltpu.VMEM((GW, C), jnp.float32),
            pltpu.VMEM((GW, C), jnp.float32),
            pltpu.VMEM((GW, 128), jnp.float32),
            pltpu.VMEM((GW, 128), jnp.float32),
            pltpu.SemaphoreType.DMA,
            pltpu.SemaphoreType.DMA,
            pltpu.SemaphoreType.DMA,
            pltpu.SemaphoreType.DMA,
            pltpu.SemaphoreType.DMA,
        ],
    )
    def k(table_hbm, lz_hbm, idx_hbm, tgt_hbm, out_hbm, parts_hbm,
          idx_v, tgt_v, lz_v, acc_v, rows0, rows1, tail0, tail1,
          g0, g1, s0, s1, lzsem):
        wid = lax.axis_index("s") * NC + lax.axis_index("c")
        base = wid * PER_W
        pltpu.sync_copy(idx_hbm.at[pl.ds(split_base + base, PER_W)], idx_v)
        pltpu.sync_copy(tgt_hbm.at[pl.ds(split_base + base, PER_W)], tgt_v)

        @pl.loop(0, PER_W, step=LZW)
        def _(j):
            sl = pl.ds(j, LZW)
            pltpu.make_async_copy(
                lz_hbm.at[idx_v.at[sl]], lz_v.at[sl], lzsem).start()

        rows = (rows0, rows1)
        tails = (tail0, tail1)
        gsem = (g0, g1)
        ssem = (s0, s1)

        def _gather_copies(c, b):
            isl = idx_v.at[pl.ds(c * GW, GW)]
            cps = []
            for t in range(7):
                cs = pl.ds(t * 128, 128)
                cps.append(pltpu.make_async_copy(
                    table_hbm.at[:, cs].at[isl], rows[b].at[:, cs], gsem[b]))
            cps.append(pltpu.make_async_copy(
                table_hbm.at[:, pl.ds(TAIL0, 128)].at[isl], tails[b], gsem[b]))
            return cps

        def _write_copies(c, b):
            dst_rows = pl.ds(base + c * GW, GW)
            cps = []
            for t in range(7):
                cs = pl.ds(t * 128, 128)
                cps.append(pltpu.make_async_copy(
                    rows[b].at[:, cs], out_hbm.at[dst_rows, cs], ssem[b]))
            ct = pl.ds(TAIL0, TAILW)
            cps.append(pltpu.make_async_copy(
                rows[b].at[:, ct], out_hbm.at[dst_rows, ct], ssem[b]))
            return cps

        def gather_start(c, b):
            for cp in _gather_copies(c, b):
                cp.start()

        def gather_wait(c, b):
            for cp in _gather_copies(c, b):
                cp.wait()

        def write_start(c, b):
            for cp in _write_copies(c, b):
                cp.start()

        def write_wait(c, b):
            for cp in _write_copies(c, b):
                cp.wait()

        def fill_tail(b):
            @pl.loop(0, GW)
            def _(r):
                for kk in (0, 16, 32, 48, 64, 80, TAILW - 16):
                    rows[b][r, pl.ds(TAIL0 + kk, 16)] = (
                        tails[b][r, pl.ds(kk, 16)])

        def extract_picked(c, b):
            for j in range(0, GW, L):
                rowi = jnp.arange(L, dtype=jnp.int32) + j
                colt = tgt_v[pl.ds(c * GW + j, L)]
                vals = plsc.load_gather(rows[b], [rowi, colt])
                acc_v[...] = acc_v[...] - vals

        gather_start(0, 0)
        gather_start(1, 1)
        acc_v[...] = jnp.zeros((L,), jnp.float32)

        @pl.loop(0, n_chunks, step=2)
        def _(c0):
            for b in range(2):
                c = c0 + b
                gather_wait(c, b)
                fill_tail(b)
                write_start(c, b)
                extract_picked(c, b)
            for b in range(2):
                nxt = c0 + 2 + b

                @pl.when(nxt < n_chunks)
                def _():
                    write_wait(c0 + b, b)
                    gather_start(nxt, b)

        @pl.loop(0, PER_W, step=LZW)
        def _(j):
            sl = pl.ds(j, LZW)
            pltpu.make_async_copy(
                lz_hbm.at[idx_v.at[sl]], lz_v.at[sl], lzsem).wait()

        @pl.loop(0, PER_W, step=L)
        def _(j):
            acc_v[...] = acc_v[...] + lz_v[pl.ds(j, L)]

        pltpu.sync_copy(acc_v, parts_hbm.at[wid])
        write_wait(n_chunks - 2, 0)
        write_wait(n_chunks - 1, 1)

    return k(table_pad, rowlz, idx_flat, tgt_flat)


def _tr_body(x_ref, o_ref):
    o_ref[...] = x_ref[...].T


def _transpose(h):
    return pl.pallas_call(
        _tr_body,
        grid=(NH // TBLK,),
        in_specs=[pl.BlockSpec((TBLK, C), lambda i: (i, 0))],
        out_specs=pl.BlockSpec((C, TBLK), lambda i: (0, i)),
        out_shape=jax.ShapeDtypeStruct((C, N), jnp.float32),
    )(h)


def _reduce_body(p_ref, o_ref):
    o_ref[...] = (jnp.sum(p_ref[...]) / N).reshape(1, 1)


def _reduce_loss(parts):
    return pl.pallas_call(
        _reduce_body,
        out_shape=jax.ShapeDtypeStruct((1, 1), jnp.float32),
    )(parts)


def kernel(idx, targets, table):
    idx_flat = idx.reshape(-1).astype(jnp.int32)
    tgt_flat = targets.reshape(-1).astype(jnp.int32)
    rowlz, table_pad = _prep(table)
    h, parts = _gather_and_parts(table_pad, rowlz, idx_flat, tgt_flat, 0)
    logits2 = _transpose(h).T
    loss = _reduce_loss(parts)
    return (logits2, loss[0, 0])
